# baseline (speedup 1.0000x reference)
import functools

import jax
import jax.numpy as jnp
from jax import lax
from jax.experimental import pallas as pl
from jax.experimental.pallas import tpu as pltpu
from jax.experimental.pallas import tpu_sc as plsc

N = 10000
E = 320000
D = 128

NC = 2
NS = 16
NW = NC * NS
EPT = E // NW
K = 80
CPT = EPT // K
RB = 80
NCHUNK = N // RB
LANES = 8


def _mm_body(x_ref, w_ref, o_ref):
    o_ref[...] = jnp.dot(x_ref[...], w_ref[...],
                         preferred_element_type=jnp.float32)


def _combine_body(p_ref, b_ref, o_ref):
    o_ref[...] = p_ref[0] + p_ref[1] + b_ref[...]


def _sc_scatter(support, dst, src, wts):
    mesh = plsc.VectorSubcoreMesh(core_axis_name="c", subcore_axis_name="s")

    @functools.partial(
        pl.kernel,
        mesh=mesh,
        out_type=jax.ShapeDtypeStruct((NC, N, D), jnp.float32),
        scratch_types=[
            pltpu.VMEM((K,), jnp.int32),
            pltpu.VMEM((K,), jnp.int32),
            pltpu.VMEM((K,), jnp.float32),
            pltpu.VMEM((K,), jnp.float32),
            pltpu.VMEM((K,), jnp.int32),
            pltpu.VMEM((K,), jnp.int32),
            pltpu.VMEM((K,), jnp.int32),
            pltpu.VMEM((K,), jnp.int32),
            pltpu.VMEM((K, D), jnp.float32),
            pltpu.VMEM((K, D), jnp.float32),
            pltpu.VMEM((K, D), jnp.float32),
            pltpu.VMEM((K, D), jnp.float32),
            pltpu.VMEM_SHARED((N, D), jnp.float32),
            pltpu.SemaphoreType.DMA,
            pltpu.SemaphoreType.DMA,
            pltpu.SemaphoreType.DMA,
            pltpu.SemaphoreType.DMA,
            pltpu.SemaphoreType.DMA,
        ],
    )
    def scatter_kernel(support_hbm, dst_hbm, src_hbm, wts_hbm, out_hbm,
                       cbuf0, cbuf1, wbuf0, wbuf1, dbuf0, dbuf1,
                       sbuf0, sbuf1, gin0, gin1, sout0, sout1, acc,
                       esem0, esem1, gsem, ssem0, ssem1):
        c = lax.axis_index("c")
        s = lax.axis_index("s")
        wid = c * NS + s

        cbuf = (cbuf0, cbuf1)
        wbuf = (wbuf0, wbuf1)
        dbuf = (dbuf0, dbuf1)
        sbuf = (sbuf0, sbuf1)
        gin = (gin0, gin1)
        sout = (sout0, sout1)
        esem = (esem0, esem1)
        ssem = (ssem0, ssem1)

        def multiply(p):
            @pl.loop(0, K // 16)
            def _(eg):
                wvec = wbuf[p][pl.ds(eg * 16, 16)]
                for l in range(16):
                    wl = jnp.broadcast_to(wvec[l], (16,))
                    e = eg * 16 + l
                    for j in range(LANES):
                        sl = pl.ds(j * 16, 16)
                        sout[p][e, sl] = gin[p][e, sl] * wl
            for i in range(K // 16):
                sl = pl.ds(i * 16, 16)
                sbuf[p][sl] = dbuf[p][sl]

        def load_edges(g, p):
            e0 = wid * EPT + g * K
            pltpu.async_copy(dst_hbm.at[pl.ds(e0, K)], dbuf[p], esem[p])
            pltpu.async_copy(src_hbm.at[pl.ds(e0, K)], cbuf[p], esem[p])
            pltpu.async_copy(wts_hbm.at[pl.ds(e0, K)], wbuf[p], esem[p])

        def wait_edges(g, p):
            e0 = wid * EPT + g * K
            pltpu.make_async_copy(dst_hbm.at[pl.ds(e0, K)], dbuf[p],
                                  esem[p]).wait()
            pltpu.make_async_copy(src_hbm.at[pl.ds(e0, K)], cbuf[p],
                                  esem[p]).wait()
            pltpu.make_async_copy(wts_hbm.at[pl.ds(e0, K)], wbuf[p],
                                  esem[p]).wait()

        def step(g, p):
            pltpu.make_async_copy(
                support_hbm.at[cbuf[p]], gin[p], gsem).wait()

            @pl.when(g + 1 < CPT)
            def _():
                wait_edges(g + 1, 1 - p)
                pltpu.async_copy(
                    support_hbm.at[cbuf[1 - p]], gin[1 - p], gsem)

            @pl.when(g > 1)
            def _():
                pltpu.make_async_copy(
                    sout[p], acc.at[sbuf[p]], ssem[p]).wait()

            multiply(p)

            @pl.when(g + 2 < CPT)
            def _():
                load_edges(g + 2, p)

            pltpu.async_copy(sout[p], acc.at[sbuf[p]], ssem[p], add=True)

        load_edges(0, 0)
        load_edges(1, 1)

        zeros16 = jnp.zeros((16,), jnp.float32)

        @pl.loop(0, RB)
        def _(i):
            for j in range(LANES):
                gin0[i, pl.ds(j * 16, 16)] = zeros16

        for i in range((NCHUNK + NS - 1) // NS):
            ck = s + i * NS

            @pl.when(ck < NCHUNK)
            def _():
                pltpu.sync_copy(gin0, acc.at[pl.ds(ck * RB, RB)])
        plsc.subcore_barrier()

        wait_edges(0, 0)
        pltpu.async_copy(support_hbm.at[cbuf0], gin0, gsem)

        @pl.loop(0, CPT, step=2)
        def _(g):
            step(g, 0)

            @pl.when(g + 1 < CPT)
            def _():
                step(g + 1, 1)

        pltpu.make_async_copy(sout1, acc.at[sbuf1], ssem1).wait()
        pltpu.make_async_copy(sout0, acc.at[sbuf0], ssem0).wait()

        plsc.subcore_barrier()

        WB = 400
        NWB = N // WB
        for i in range((NWB + NS - 1) // NS):
            ck = s + i * NS

            @pl.when(ck < NWB)
            def _():
                pltpu.sync_copy(acc.at[pl.ds(ck * WB, WB)],
                                out_hbm.at[c, pl.ds(ck * WB, WB)])

    return scatter_kernel(support, dst, src, wts)


def kernel(x, edge_index, edge_weight, W, b):
    support = pl.pallas_call(
        _mm_body,
        out_shape=jax.ShapeDtypeStruct((N, D), jnp.float32),
    )(x, W)

    dst = edge_index[0]
    src = edge_index[1]
    wts = edge_weight

    partial = _sc_scatter(support, dst, src, wts)

    out = pl.pallas_call(
        _combine_body,
        out_shape=jax.ShapeDtypeStruct((N, D), jnp.float32),
    )(partial, b.reshape(1, D))
    return out


# speedup vs baseline: 1.1003x; 1.1003x over previous
import functools

import jax
import jax.numpy as jnp
from jax import lax
from jax.experimental import pallas as pl
from jax.experimental.pallas import tpu as pltpu
from jax.experimental.pallas import tpu_sc as plsc

N = 10000
E = 320000
D = 128

NC = 2
NS = 16
NW = NC * NS
EPT = E // NW
K = 80
CPT = EPT // K
RB = 80
NCHUNK = N // RB
LANES = 8


def _mm_body(x_ref, w_ref, o_ref):
    o_ref[...] = jnp.dot(x_ref[...], w_ref[...],
                         preferred_element_type=jnp.float32)


def _combine_body(p_ref, b_ref, o_ref):
    o_ref[...] = p_ref[0] + p_ref[1] + b_ref[...]


def _sc_scatter(support, dst, src, wts):
    mesh = plsc.VectorSubcoreMesh(core_axis_name="c", subcore_axis_name="s")

    @functools.partial(
        pl.kernel,
        mesh=mesh,
        out_type=jax.ShapeDtypeStruct((NC, N, D), jnp.float32),
        scratch_types=[
            pltpu.VMEM((EPT,), jnp.int32),
            pltpu.VMEM((EPT,), jnp.int32),
            pltpu.VMEM((K,), jnp.float32),
            pltpu.VMEM((K,), jnp.float32),
            pltpu.VMEM((K,), jnp.int32),
            pltpu.VMEM((K,), jnp.int32),
            pltpu.VMEM((K, D), jnp.float32),
            pltpu.VMEM((K, D), jnp.float32),
            pltpu.VMEM_SHARED((N, D), jnp.float32),
            pltpu.SemaphoreType.DMA,
            pltpu.SemaphoreType.DMA,
            pltpu.SemaphoreType.DMA,
            pltpu.SemaphoreType.DMA,
        ],
    )
    def scatter_kernel(support_hbm, dst_hbm, src_hbm, wts_hbm, out_hbm,
                       srcb, dstb, wbuf0, wbuf1, sbuf0, sbuf1,
                       rows0, rows1, acc, esem0, esem1, gsem, ssem):
        c = lax.axis_index("c")
        s = lax.axis_index("s")
        wid = c * NS + s

        wbuf = (wbuf0, wbuf1)
        sbuf = (sbuf0, sbuf1)
        rows = (rows0, rows1)
        esem = (esem0, esem1)

        def multiply(gi, p):
            @pl.loop(0, K // 16)
            def _(eg):
                wvec = wbuf[p][pl.ds(eg * 16, 16)]
                for l in range(16):
                    wl = jnp.broadcast_to(wvec[l], (16,))
                    e = eg * 16 + l
                    for j in range(LANES):
                        sl = pl.ds(j * 16, 16)
                        rows[p][e, sl] = rows[p][e, sl] * wl
            for i in range(K // 16):
                sbuf[p][pl.ds(i * 16, 16)] = dstb[pl.ds(gi * K + i * 16, 16)]

        def load_edges(g, p):
            e0 = wid * EPT + g * K
            pltpu.async_copy(wts_hbm.at[pl.ds(e0, K)], wbuf[p], esem[p])

        def wait_edges(g, p):
            e0 = wid * EPT + g * K
            pltpu.make_async_copy(wts_hbm.at[pl.ds(e0, K)], wbuf[p],
                                  esem[p]).wait()

        def step(g, p):
            pltpu.make_async_copy(
                support_hbm.at[srcb.at[pl.ds(g * K, K)]], rows[p],
                gsem).wait()

            @pl.when(g + 1 < CPT)
            def _():
                wait_edges(g + 1, 1 - p)

            @pl.when(g > 0)
            def _():
                pltpu.make_async_copy(
                    rows[1 - p], acc.at[sbuf[1 - p]], ssem).wait()

            @pl.when(g + 1 < CPT)
            def _():
                pltpu.async_copy(
                    support_hbm.at[srcb.at[pl.ds((g + 1) * K, K)]],
                    rows[1 - p], gsem)

            multiply(g, p)

            @pl.when(g + 2 < CPT)
            def _():
                load_edges(g + 2, p)

            pltpu.async_copy(rows[p], acc.at[sbuf[p]], ssem, add=True)

        base = wid * EPT
        pltpu.async_copy(src_hbm.at[pl.ds(base, EPT)], srcb, gsem)
        pltpu.async_copy(dst_hbm.at[pl.ds(base, EPT)], dstb, gsem)
        load_edges(0, 0)
        load_edges(1, 1)

        zeros16 = jnp.zeros((16,), jnp.float32)

        @pl.loop(0, RB)
        def _(i):
            for j in range(LANES):
                rows0[i, pl.ds(j * 16, 16)] = zeros16

        for i in range((NCHUNK + NS - 1) // NS):
            ck = s + i * NS

            @pl.when(ck < NCHUNK)
            def _():
                pltpu.sync_copy(rows0, acc.at[pl.ds(ck * RB, RB)])
        plsc.subcore_barrier()

        pltpu.make_async_copy(src_hbm.at[pl.ds(base, EPT)], srcb,
                              gsem).wait()
        pltpu.make_async_copy(dst_hbm.at[pl.ds(base, EPT)], dstb,
                              gsem).wait()
        wait_edges(0, 0)
        pltpu.async_copy(support_hbm.at[srcb.at[pl.ds(0, K)]], rows0, gsem)

        @pl.loop(0, CPT, step=2)
        def _(g):
            step(g, 0)

            @pl.when(g + 1 < CPT)
            def _():
                step(g + 1, 1)

        pltpu.make_async_copy(rows0, acc.at[sbuf0], ssem).wait()

        plsc.subcore_barrier()

        WB = 400
        NWB = N // WB
        for i in range((NWB + NS - 1) // NS):
            ck = s + i * NS

            @pl.when(ck < NWB)
            def _():
                pltpu.sync_copy(acc.at[pl.ds(ck * WB, WB)],
                                out_hbm.at[c, pl.ds(ck * WB, WB)])

    return scatter_kernel(support, dst, src, wts)


def kernel(x, edge_index, edge_weight, W, b):
    support = pl.pallas_call(
        _mm_body,
        out_shape=jax.ShapeDtypeStruct((N, D), jnp.float32),
    )(x, W)

    dst = edge_index[0]
    src = edge_index[1]
    wts = edge_weight

    partial = _sc_scatter(support, dst, src, wts)

    out = pl.pallas_call(
        _combine_body,
        out_shape=jax.ShapeDtypeStruct((N, D), jnp.float32),
    )(partial, b.reshape(1, D))
    return out
